# Initial kernel scaffold; baseline (speedup 1.0000x reference)
import jax
import jax.numpy as jnp
from jax.experimental import pallas as pl


def kernel(x, Wr, br, W1, b1, W2, b2):
    raise NotImplementedError("write your pallas kernel here")


# speedup vs baseline: 7.1188x; 7.1188x over previous
import functools

import jax
import jax.numpy as jnp
from jax import lax
from jax.experimental import pallas as pl
from jax.experimental.pallas import tpu as pltpu
from jax.experimental.pallas import tpu_sc as plsc

D = 768
E = 64
T = 2048
H = 4 * D
TT = 64
NW = 32
BPW = T // NW


def _router_body(x_ref, wr_ref, br_ref, pos_ref, starts_ref, aux_ref, z_ref):
    x = x_ref[...]
    logits = lax.dot_general(
        x, wr_ref[...], (((1,), (1,)), ((), ())),
        preferred_element_type=jnp.float32) + br_ref[...][None, :]
    m = jnp.max(logits, axis=1, keepdims=True)
    ecol = lax.broadcasted_iota(jnp.int32, (T, E), 1)
    eid = jnp.min(jnp.where(logits == m, ecol, E), axis=1, keepdims=True)
    onehot = (ecol == eid).astype(jnp.float32)

    z = m[:, 0] + jnp.log(jnp.sum(jnp.exp(logits - m), axis=1))
    z_ref[...] = jnp.mean(z * z).reshape(1, 1)

    counts = jnp.sum(onehot, axis=0, keepdims=True)
    frac = counts * (1.0 / T)
    aux_ref[...] = (E * jnp.sum(frac * frac)).reshape(1, 1)

    r_e = lax.broadcasted_iota(jnp.int32, (E, E + 1), 0)
    c_e = lax.broadcasted_iota(jnp.int32, (E, E + 1), 1)
    starts = lax.dot_general(
        counts, (r_e < c_e).astype(jnp.float32), (((1,), (0,)), ((), ())),
        preferred_element_type=jnp.float32)
    starts_ref[...] = starts.astype(jnp.int32)

    C = 256
    r_c = lax.broadcasted_iota(jnp.int32, (C, C), 0)
    c_c = lax.broadcasted_iota(jnp.int32, (C, C), 1)
    tri = (c_c < r_c).astype(jnp.float32)
    sbase = starts[:, :E]
    acc = jnp.zeros((1, E), jnp.float32)
    for j in range(T // C):
        oh_j = onehot[j * C:(j + 1) * C, :]
        prior_j = lax.dot_general(
            tri, oh_j, (((1,), (0,)), ((), ())),
            preferred_element_type=jnp.float32) + acc
        posf_j = jnp.sum(oh_j * (sbase + prior_j), axis=1)
        pos_ref[j * C:(j + 1) * C, :] = posf_j.astype(jnp.int32)[:, None]
        acc = acc + jnp.sum(oh_j, axis=0, keepdims=True)


def _router(flat_x, wr, br):
    return pl.pallas_call(
        _router_body,
        out_shape=(
            jax.ShapeDtypeStruct((T, 1), jnp.int32),
            jax.ShapeDtypeStruct((1, E + 1), jnp.int32),
            jax.ShapeDtypeStruct((1, 1), jnp.float32),
            jax.ShapeDtypeStruct((1, 1), jnp.float32),
        ),
    )(flat_x, wr, br)


def _ffn_body(starts_ref, xs_ref, w1_ref, b1_ref, w2_ref, b2_ref, out_ref):
    e = pl.program_id(0)
    start = starts_ref[e]
    end = starts_ref[e + 1]
    astart = (start // 8) * 8
    nt = (end - astart + TT - 1) // TT

    def body(i, carry):
        base = pl.multiple_of(jnp.minimum(astart + i * TT, T - TT), 8)
        rows = xs_ref[pl.ds(base, TT), :]
        h = lax.dot_general(rows, w1_ref[0], (((1,), (1,)), ((), ())),
                            preferred_element_type=jnp.float32) + b1_ref[0]
        h = jnp.maximum(h, 0.0)
        o = lax.dot_general(h, w2_ref[0], (((1,), (1,)), ((), ())),
                            preferred_element_type=jnp.float32) + b2_ref[0]
        rid = base + lax.broadcasted_iota(jnp.int32, (TT, 1), 0)
        mask = (rid >= start) & (rid < end)
        old = out_ref[pl.ds(base, TT), :]
        out_ref[pl.ds(base, TT), :] = jnp.where(mask, o, old)
        return carry

    lax.fori_loop(0, nt, body, 0)


def _ffn(starts, xs, w1, b1, w2, b2):
    return pl.pallas_call(
        _ffn_body,
        grid=(E,),
        in_specs=[
            pl.BlockSpec(memory_space=pltpu.SMEM),
            pl.BlockSpec((T, D), lambda e: (0, 0)),
            pl.BlockSpec((1, H, D), lambda e: (e, 0, 0)),
            pl.BlockSpec((1, 1, H), lambda e: (e, 0, 0)),
            pl.BlockSpec((1, D, H), lambda e: (e, 0, 0)),
            pl.BlockSpec((1, 1, D), lambda e: (e, 0, 0)),
        ],
        out_specs=pl.BlockSpec((T, D), lambda e: (0, 0)),
        out_shape=jax.ShapeDtypeStruct((T, D), jnp.float32),
        compiler_params=pltpu.CompilerParams(
            dimension_semantics=("arbitrary",),
            vmem_limit_bytes=100 * 1024 * 1024),
    )(starts, xs, w1, b1.reshape(E, 1, H), w2, b2.reshape(E, 1, D))


def _sc_mesh():
    return plsc.VectorSubcoreMesh(core_axis_name="c", subcore_axis_name="s")


@functools.cache
def _sc_scatter():
    @functools.partial(
        pl.kernel,
        out_type=jax.ShapeDtypeStruct((T, D), jnp.float32),
        mesh=_sc_mesh(),
        scratch_types=[
            pltpu.VMEM((BPW,), jnp.int32),
            pltpu.VMEM((BPW, D), jnp.float32),
            pltpu.SemaphoreType.DMA,
        ],
    )
    def k(x_hbm, pos_hbm, out_hbm, idx_v, rows_v, sem):
        wid = lax.axis_index("s") * 2 + lax.axis_index("c")
        base = wid * BPW
        pltpu.sync_copy(pos_hbm.at[pl.ds(base, BPW)], idx_v)
        pltpu.sync_copy(x_hbm.at[pl.ds(base, BPW)], rows_v)
        pltpu.async_copy(rows_v, out_hbm.at[idx_v], sem).wait()

    return k


@functools.cache
def _sc_gather():
    @functools.partial(
        pl.kernel,
        out_type=jax.ShapeDtypeStruct((T, D), jnp.float32),
        mesh=_sc_mesh(),
        scratch_types=[
            pltpu.VMEM((BPW,), jnp.int32),
            pltpu.VMEM((BPW, D), jnp.float32),
            pltpu.SemaphoreType.DMA,
        ],
    )
    def k(src_hbm, pos_hbm, out_hbm, idx_v, rows_v, sem):
        wid = lax.axis_index("s") * 2 + lax.axis_index("c")
        base = wid * BPW
        pltpu.sync_copy(pos_hbm.at[pl.ds(base, BPW)], idx_v)
        pltpu.async_copy(src_hbm.at[idx_v], rows_v, sem).wait()
        pltpu.sync_copy(rows_v, out_hbm.at[pl.ds(base, BPW)])

    return k


def kernel(x, Wr, br, W1, b1, W2, b2):
    flat_x = x.reshape(T, D)
    pos2d, starts2d, aux, z = _router(flat_x, Wr, br)
    pos = pos2d.reshape(T)
    starts = starts2d.reshape(E + 1)
    xs = _sc_scatter()(flat_x, pos)
    outs = _ffn(starts, xs, W1, b1, W2, b2)
    final = _sc_gather()(outs, pos)
    return final.reshape(x.shape), aux.reshape(()), z.reshape(())
